# baseline (speedup 1.0000x reference)
import functools
import math

import jax
import jax.numpy as jnp
import numpy as np
from jax import lax
from jax.experimental import pallas as pl
from jax.experimental.pallas import tpu as pltpu
from jax.experimental.pallas import tpu_sc as plsc

_INTERPRET = False

_B = 8
_N = 2048
_S = 512
_K = 64
_EPS = 1e-5
_R2 = np.float32(0.2 ** 2)
_M = _B * _S * _K


def _fps_body(x_ref, y_ref, z_ref, cx_ref, cy_ref, cz_ref):
    x = x_ref[...]
    y = y_ref[...]
    z = z_ref[...]
    iota = lax.broadcasted_iota(jnp.int32, (_B, _N), 1)
    iota_s = lax.broadcasted_iota(jnp.int32, (_B, _S), 1)

    def body(i, carry):
        dist, far, cxs, cys, czs = carry
        m = iota == far
        cx = jnp.sum(jnp.where(m, x, 0.0), axis=1, keepdims=True)
        cy = jnp.sum(jnp.where(m, y, 0.0), axis=1, keepdims=True)
        cz = jnp.sum(jnp.where(m, z, 0.0), axis=1, keepdims=True)
        sm = iota_s == i
        cxs = jnp.where(sm, cx, cxs)
        cys = jnp.where(sm, cy, cys)
        czs = jnp.where(sm, cz, czs)
        dx = x - cx
        dy = y - cy
        dz = z - cz
        d = (dx * dx + dy * dy) + dz * dz
        dist = jnp.minimum(dist, d)
        mx = jnp.max(dist, axis=1, keepdims=True)
        far = jnp.min(jnp.where(dist == mx, iota, _N), axis=1, keepdims=True)
        return dist, far, cxs, cys, czs

    dist0 = jnp.full((_B, _N), 1e10, jnp.float32)
    far0 = jnp.zeros((_B, 1), jnp.int32)
    zs0 = jnp.zeros((_B, _S), jnp.float32)
    _, _, cxs, cys, czs = lax.fori_loop(0, _S, body,
                                        (dist0, far0, zs0, zs0, zs0))
    cx_ref[...] = cxs
    cy_ref[...] = cys
    cz_ref[...] = czs


def _run_fps(xs, ys, zs):
    return pl.pallas_call(
        _fps_body,
        out_shape=(
            jax.ShapeDtypeStruct((_B, _S), jnp.float32),
            jax.ShapeDtypeStruct((_B, _S), jnp.float32),
            jax.ShapeDtypeStruct((_B, _S), jnp.float32),
        ),
        interpret=_INTERPRET,
    )(xs, ys, zs)


def _dense_body(pts_ref, xyzf_ref, nxyzf_ref, w0p_ref, w0x_ref, b0_ref,
                a_ref, c_ref):
    a = lax.dot_general(pts_ref[...], w0p_ref[...], (((1,), (0,)), ((), ())),
                        precision=lax.Precision.HIGHEST,
                        preferred_element_type=jnp.float32)
    a = a + lax.dot_general(xyzf_ref[...], w0x_ref[...],
                            (((1,), (0,)), ((), ())),
                            precision=lax.Precision.HIGHEST,
                            preferred_element_type=jnp.float32)
    a_ref[...] = a + b0_ref[...]
    c_ref[...] = lax.dot_general(nxyzf_ref[...], w0x_ref[...],
                                 (((1,), (0,)), ((), ())),
                                 precision=lax.Precision.HIGHEST,
                                 preferred_element_type=jnp.float32)


def _run_dense(points_flat, xyz_flat, nxyz_flat, w0p_pad, w0x_pad, b0_pad,
               tiles=16):
    ra = _B * _N // tiles
    rc = _B * _S // tiles
    return pl.pallas_call(
        _dense_body,
        grid=(tiles,),
        in_specs=[
            pl.BlockSpec((ra, 128), lambda t: (t, 0)),
            pl.BlockSpec((ra, 8), lambda t: (t, 0)),
            pl.BlockSpec((rc, 8), lambda t: (t, 0)),
            pl.BlockSpec((128, 128), lambda t: (0, 0)),
            pl.BlockSpec((8, 128), lambda t: (0, 0)),
            pl.BlockSpec((1, 128), lambda t: (0, 0)),
        ],
        out_specs=(
            pl.BlockSpec((ra, 128), lambda t: (t, 0)),
            pl.BlockSpec((rc, 128), lambda t: (t, 0)),
        ),
        out_shape=(
            jax.ShapeDtypeStruct((_B * _N, 128), jnp.float32),
            jax.ShapeDtypeStruct((_B * _S, 128), jnp.float32),
        ),
        interpret=_INTERPRET,
    )(points_flat, xyz_flat, nxyz_flat, w0p_pad, w0x_pad, b0_pad)


def _sel_body(nxyz_ref, xyz_ref, a_ref, c_ref, tri_ref, spos_ref, aux_ref,
              st1_ref, acc):
    t = pl.program_id(0)
    a = nxyz_ref[0]
    b = xyz_ref[0]
    mm = lax.dot_general(a, b, (((1,), (1,)), ((), ())),
                         preferred_element_type=jnp.float32)
    d = -2.0 * mm
    d = d + jnp.sum(a * a, -1)[:, None]
    d = d + jnp.sum(b * b, -1)[None, :]
    inr = jnp.logical_not(d > _R2)
    maskf = jnp.where(inr, 1.0, 0.0).astype(jnp.float32)
    tri = tri_ref[...]
    lane = lax.broadcasted_iota(jnp.int32, (_S, 128), 1)
    lanef = lane.astype(jnp.float32)

    base = jnp.zeros((_S, 1), jnp.float32)
    first = jnp.full((_S, 1), jnp.float32(_N * 2))
    poss = []
    for j in range(16):
        mj = maskf[:, j * 128:(j + 1) * 128]
        pos_j = lax.dot_general(mj, tri, (((1,), (0,)), ((), ())),
                                preferred_element_type=jnp.float32) + base
        poss.append(pos_j)
        base = base + jnp.sum(mj, axis=1, keepdims=True)
        nval = lanef + jnp.float32(j * 128)
        first = jnp.minimum(
            first, jnp.min(jnp.where(mj > 0.0, nval, jnp.float32(_N * 2)),
                           axis=1, keepdims=True))
    cnt = base
    selm_parts = []
    trash = jnp.float32(_K) + (lane & 15).astype(jnp.float32)
    for j in range(16):
        mj = maskf[:, j * 128:(j + 1) * 128]
        validj = (mj > 0.0) & (poss[j] < jnp.float32(_K))
        spos_ref[:, j, :] = jnp.where(validj, poss[j], trash).astype(jnp.int32)
        selm_parts.append(jnp.where(validj, 1.0, 0.0).astype(jnp.float32))
    selm = jnp.concatenate(selm_parts, axis=1)

    padw = jnp.maximum(jnp.float32(_K) - cnt, 0.0)
    iota_n = lax.broadcasted_iota(jnp.int32, (_S, _N), 1).astype(jnp.float32)
    padoh = jnp.where(iota_n == first, padw, 0.0)
    selmtot = selm + padoh

    aux = (jnp.where(lane == 0, cnt.astype(jnp.int32), 0)
           + jnp.where(lane == 1, first.astype(jnp.int32), 0))
    aux_ref[...] = aux

    av = a_ref[...]
    cv = c_ref[...]
    h = lax.dot_general(selmtot, av, (((1,), (0,)), ((), ())),
                        preferred_element_type=jnp.float32)
    co = jnp.sum(selmtot, axis=0, keepdims=True)
    coa2 = lax.dot_general(co, av * av, (((1,), (0,)), ((), ())),
                           preferred_element_type=jnp.float32)[0]
    s1 = jnp.sum(h, axis=0) - jnp.float32(_K) * jnp.sum(cv, axis=0)
    s2 = (coa2 - 2.0 * jnp.sum(cv * h, axis=0)
          + jnp.float32(_K) * jnp.sum(cv * cv, axis=0))
    st = jnp.stack([s1, s2])

    @pl.when(t == 0)
    def _init():
        acc[...] = st

    @pl.when(t > 0)
    def _accum():
        acc[...] = acc[...] + st

    @pl.when(t == pl.num_programs(0) - 1)
    def _emit():
        st1_ref[...] = acc[...]


def _run_select(new_xyz, xyz, a_pad, c_pad, tri):
    return pl.pallas_call(
        _sel_body,
        grid=(_B,),
        in_specs=[pl.BlockSpec((1, _S, 3), lambda i: (i, 0, 0)),
                  pl.BlockSpec((1, _N, 3), lambda i: (i, 0, 0)),
                  pl.BlockSpec((_N, 128), lambda i: (i, 0)),
                  pl.BlockSpec((_S, 128), lambda i: (i, 0)),
                  pl.BlockSpec((128, 128), lambda i: (0, 0))],
        out_specs=(
            pl.BlockSpec((_S, 16, 128), lambda i: (i, 0, 0)),
            pl.BlockSpec((_S, 128), lambda i: (i, 0)),
            pl.BlockSpec((2, 128), lambda i: (0, 0)),
        ),
        out_shape=(
            jax.ShapeDtypeStruct((_B * _S, 16, 128), jnp.int32),
            jax.ShapeDtypeStruct((_B * _S, 128), jnp.int32),
            jax.ShapeDtypeStruct((2, 128), jnp.float32),
        ),
        scratch_shapes=[pltpu.VMEM((2, 128), jnp.float32)],
        interpret=_INTERPRET,
    )(new_xyz, xyz, a_pad, c_pad, tri)


def _stats1_body(g_ref, c_ref, o_ref, acc):
    t = pl.program_id(0)
    z = g_ref[...] - c_ref[...][:, None, :]
    s1 = jnp.sum(z, axis=(0, 1))
    s2 = jnp.sum(z * z, axis=(0, 1))
    st = jnp.stack([s1, s2])

    @pl.when(t == 0)
    def _():
        acc[...] = st

    @pl.when(t > 0)
    def _():
        acc[...] = acc[...] + st

    @pl.when(t == pl.num_programs(0) - 1)
    def _():
        o_ref[...] = acc[...]


def _run_stats1(g, c, tiles=32):
    rows = _B * _S // tiles
    return pl.pallas_call(
        _stats1_body,
        grid=(tiles,),
        in_specs=[
            pl.BlockSpec((rows, _K, 128), lambda t: (t, 0, 0)),
            pl.BlockSpec((rows, 128), lambda t: (t, 0)),
        ],
        out_specs=pl.BlockSpec((2, 128), lambda t: (0, 0)),
        out_shape=jax.ShapeDtypeStruct((2, 128), jnp.float32),
        scratch_shapes=[pltpu.VMEM((2, 128), jnp.float32)],
        interpret=_INTERPRET,
    )(g, c)


def _mlp_body(g_ref, c_ref, st1_ref, w1_ref, w2_ref, bg1_ref, bg2_ref,
              out_ref, acc2, acc3, mx_ref):
    p = pl.program_id(0)
    t = pl.program_id(1)
    nt = pl.num_programs(1)
    st = st1_ref[...]
    mean1 = st[0] * (1.0 / _M)
    var1 = st[1] * (1.0 / _M) - mean1 * mean1
    inv1 = bg1_ref[2] / jnp.sqrt(var1 + _EPS)
    z1 = g_ref[...] - c_ref[...][:, None, :]
    x1 = jnp.maximum((z1 - mean1[None, None, :]) * inv1[None, None, :]
                     + bg1_ref[3][None, None, :], 0.0)
    rows = x1.shape[0] * x1.shape[1]
    z2 = lax.dot_general(x1.reshape(rows, 128), w1_ref[...],
                         (((1,), (0,)), ((), ())),
                         precision=lax.Precision.HIGHEST,
                         preferred_element_type=jnp.float32)
    z2 = z2 + bg1_ref[0][None, :]

    @pl.when(p == 0)
    def _phase_stats():
        s1 = jnp.sum(z2, axis=0)
        s2 = jnp.sum(z2 * z2, axis=0)
        st_new = jnp.stack([s1, s2])

        @pl.when(t == 0)
        def _():
            acc2[...] = st_new

        @pl.when(t > 0)
        def _():
            acc2[...] = acc2[...] + st_new

    @pl.when(p == 1)
    def _phase_layer3():
        st2 = acc2[...]
        mean2 = st2[0] * (1.0 / _M)
        var2 = st2[1] * (1.0 / _M) - mean2 * mean2
        inv2 = bg2_ref[2] / jnp.sqrt(var2 + _EPS)
        x2 = jnp.maximum((z2 - mean2[None, :]) * inv2[None, :]
                         + bg2_ref[3][None, :], 0.0)
        z3 = lax.dot_general(x2, w2_ref[...], (((1,), (0,)), ((), ())),
                             precision=lax.Precision.HIGHEST,
                             preferred_element_type=jnp.float32)
        z3 = z3 + bg2_ref[0][None, :]
        s1 = jnp.sum(z3, axis=0)
        s2 = jnp.sum(z3 * z3, axis=0)
        st_new = jnp.stack([s1, s2])

        @pl.when(t == 0)
        def _():
            acc3[...] = st_new

        @pl.when(t > 0)
        def _():
            acc3[...] = acc3[...] + st_new

        srows = z3.shape[0] // _K
        mx = jnp.max(z3.reshape(srows, _K, 128), axis=1)
        mx_ref[pl.ds(t * srows, srows), :] = mx

        @pl.when(t == nt - 1)
        def _():
            stf = acc3[...]
            mean3 = stf[0] * (1.0 / _M)
            var3 = stf[1] * (1.0 / _M) - mean3 * mean3
            inv3 = bg2_ref[6] / jnp.sqrt(var3 + _EPS)
            pooled = mx_ref[...]
            out_ref[...] = jnp.maximum(
                (pooled - mean3[None, :]) * inv3[None, :]
                + bg2_ref[7][None, :], 0.0)


def _run_mlp(g, c, st1, w1_pad, w2_pad, bg1, bg2, tiles=32):
    rows = _B * _S // tiles
    return pl.pallas_call(
        _mlp_body,
        grid=(2, tiles),
        in_specs=[
            pl.BlockSpec((rows, _K, 128), lambda p, t: (t, 0, 0)),
            pl.BlockSpec((rows, 128), lambda p, t: (t, 0)),
            pl.BlockSpec((2, 128), lambda p, t: (0, 0)),
            pl.BlockSpec((128, 128), lambda p, t: (0, 0)),
            pl.BlockSpec((128, 128), lambda p, t: (0, 0)),
            pl.BlockSpec((4, 128), lambda p, t: (0, 0)),
            pl.BlockSpec((8, 128), lambda p, t: (0, 0)),
        ],
        out_specs=pl.BlockSpec((_B * _S, 128), lambda p, t: (0, 0)),
        out_shape=jax.ShapeDtypeStruct((_B * _S, 128), jnp.float32),
        scratch_shapes=[
            pltpu.VMEM((2, 128), jnp.float32),
            pltpu.VMEM((2, 128), jnp.float32),
            pltpu.VMEM((_B * _S, 128), jnp.float32),
        ],
        interpret=_INTERPRET,
    )(g, c, st1, w1_pad, w2_pad, bg1, bg2)


def _layer2_body(g_ref, c_ref, st1_ref, w1_ref, bg_ref, z2_ref, st2_ref, acc):
    t = pl.program_id(0)
    st = st1_ref[...]
    mean = st[0] * (1.0 / _M)
    var = st[1] * (1.0 / _M) - mean * mean
    inv = bg_ref[2] / jnp.sqrt(var + _EPS)
    z = g_ref[...] - c_ref[...][:, None, :]
    x1 = jnp.maximum((z - mean[None, None, :]) * inv[None, None, :]
                     + bg_ref[3][None, None, :], 0.0)
    rows = x1.shape[0] * x1.shape[1]
    z2 = lax.dot_general(x1.reshape(rows, 128), w1_ref[...],
                         (((1,), (0,)), ((), ())),
                         precision=lax.Precision.HIGHEST,
                         preferred_element_type=jnp.float32)
    z2 = z2 + bg_ref[0][None, :]
    z2_ref[...] = z2
    s1 = jnp.sum(z2, axis=0)
    s2 = jnp.sum(z2 * z2, axis=0)
    st_new = jnp.stack([s1, s2])

    @pl.when(t == 0)
    def _():
        acc[...] = st_new

    @pl.when(t > 0)
    def _():
        acc[...] = acc[...] + st_new

    @pl.when(t == pl.num_programs(0) - 1)
    def _():
        st2_ref[...] = acc[...]


def _run_layer2(g, c, st1, w1_pad, bg1, tiles=32):
    rows = _B * _S // tiles
    return pl.pallas_call(
        _layer2_body,
        grid=(tiles,),
        in_specs=[
            pl.BlockSpec((rows, _K, 128), lambda t: (t, 0, 0)),
            pl.BlockSpec((rows, 128), lambda t: (t, 0)),
            pl.BlockSpec((2, 128), lambda t: (0, 0)),
            pl.BlockSpec((128, 128), lambda t: (0, 0)),
            pl.BlockSpec((4, 128), lambda t: (0, 0)),
        ],
        out_specs=(
            pl.BlockSpec((rows * _K, 128), lambda t: (t, 0)),
            pl.BlockSpec((2, 128), lambda t: (0, 0)),
        ),
        out_shape=(
            jax.ShapeDtypeStruct((_B * _S * _K, 128), jnp.float32),
            jax.ShapeDtypeStruct((2, 128), jnp.float32),
        ),
        scratch_shapes=[pltpu.VMEM((2, 128), jnp.float32)],
        interpret=_INTERPRET,
    )(g, c, st1, w1_pad, bg1)


def _layer3_body(z2_ref, st2_ref, w2_ref, bg_ref, out_ref, acc, mx_ref):
    t = pl.program_id(0)
    nt = pl.num_programs(0)
    st = st2_ref[...]
    mean = st[0] * (1.0 / _M)
    var = st[1] * (1.0 / _M) - mean * mean
    inv = bg_ref[2] / jnp.sqrt(var + _EPS)
    x2 = jnp.maximum((z2_ref[...] - mean[None, :]) * inv[None, :]
                     + bg_ref[3][None, :], 0.0)
    z3 = lax.dot_general(x2, w2_ref[...], (((1,), (0,)), ((), ())),
                         precision=lax.Precision.HIGHEST,
                         preferred_element_type=jnp.float32)
    z3 = z3 + bg_ref[0][None, :]
    s1 = jnp.sum(z3, axis=0)
    s2 = jnp.sum(z3 * z3, axis=0)
    st_new = jnp.stack([s1, s2])

    @pl.when(t == 0)
    def _():
        acc[...] = st_new

    @pl.when(t > 0)
    def _():
        acc[...] = acc[...] + st_new

    rows = z3.shape[0] // _K
    mx = jnp.max(z3.reshape(rows, _K, 128), axis=1)
    mx_ref[pl.ds(t * rows, rows), :] = mx

    @pl.when(t == nt - 1)
    def _():
        stf = acc[...]
        mean3 = stf[0] * (1.0 / _M)
        var3 = stf[1] * (1.0 / _M) - mean3 * mean3
        inv3 = bg_ref[6] / jnp.sqrt(var3 + _EPS)
        pooled = mx_ref[...]
        out_ref[...] = jnp.maximum(
            (pooled - mean3[None, :]) * inv3[None, :] + bg_ref[7][None, :], 0.0)


def _run_layer3(z2, st2, w2_pad, bg2, tiles=32):
    rows = _B * _S * _K // tiles
    return pl.pallas_call(
        _layer3_body,
        grid=(tiles,),
        in_specs=[
            pl.BlockSpec((rows, 128), lambda t: (t, 0)),
            pl.BlockSpec((2, 128), lambda t: (0, 0)),
            pl.BlockSpec((128, 128), lambda t: (0, 0)),
            pl.BlockSpec((8, 128), lambda t: (0, 0)),
        ],
        out_specs=pl.BlockSpec((_B * _S, 128), lambda t: (0, 0)),
        out_shape=jax.ShapeDtypeStruct((_B * _S, 128), jnp.float32),
        scratch_shapes=[
            pltpu.VMEM((2, 128), jnp.float32),
            pltpu.VMEM((_B * _S, 128), jnp.float32),
        ],
        interpret=_INTERPRET,
    )(z2, st2, w2_pad, bg2)


_NW = 32
_CPW = (_B * _S) // _NW


_NBUF = 4


def _sc_body(spos_hbm, aux_hbm, a_hbm, g_hbm, *refs):
    drows = refs[0:4]
    gbufs = refs[4:8]
    rowss = refs[8:12]
    aux_v = refs[12]
    selbuf = refs[13]
    dsems = refs[14:18]
    gsems = refs[18:22]
    osems = refs[22:26]
    cid = lax.axis_index("c")
    sid = lax.axis_index("s")
    wid = sid * 2 + cid
    base = wid * _CPW
    iota16 = lax.iota(jnp.int32, 16)

    pltpu.sync_copy(aux_hbm.at[pl.ds(base * 128, _CPW * 128)], aux_v)

    def select(drow, gbuf, i, s):
        for r in range(16):
            for l in range(8):
                sp16 = drow[r, pl.ds(l * 16, 16)]
                vals = iota16 + (r * 128 + l * 16)
                plsc.store_scatter(selbuf, [sp16], vals)
        v16 = aux_v[pl.ds(i * 128, 16)]
        cnt = lax.squeeze(lax.slice(v16, (0,), (1,)), (0,))
        first = lax.squeeze(lax.slice(v16, (1,), (2,)), (0,))
        off = lax.shift_right_logical(s, 9) * _N
        for q in range(4):
            cur = selbuf[pl.ds(q * 16, 16)]
            sel = jnp.where((iota16 + q * 16) < cnt, cur, first)
            gbuf[pl.ds(q * 16, 16)] = sel + off

    for p in range(_NBUF - 1):
        pltpu.make_async_copy(spos_hbm.at[base + p], drows[p],
                              dsems[p]).start()

    def outer(gi, carry):
        for par in range(_NBUF):
            i = gi * _NBUF + par
            s = base + i
            pltpu.make_async_copy(spos_hbm.at[s], drows[par],
                                  dsems[par]).wait()

            @pl.when(i + _NBUF - 1 < _CPW)
            def _start_next_dist():
                nx = (par + _NBUF - 1) % _NBUF
                pltpu.make_async_copy(spos_hbm.at[s + _NBUF - 1],
                                      drows[nx], dsems[nx]).start()

            select(drows[par], gbufs[par], i, s)

            @pl.when(i >= _NBUF)
            def _wait_out_copy():
                pltpu.make_async_copy(rowss[par], g_hbm.at[s - _NBUF],
                                      osems[par]).wait()

            pltpu.make_async_copy(a_hbm.at[gbufs[par]], rowss[par],
                                  gsems[par]).start()

            @pl.when(i >= _NBUF - 1)
            def _drain_old_gather():
                pv = (par + 1) % _NBUF
                pltpu.make_async_copy(a_hbm.at[gbufs[pv]],
                                      rowss[pv], gsems[pv]).wait()
                pltpu.make_async_copy(rowss[pv], g_hbm.at[s - (_NBUF - 1)],
                                      osems[pv]).start()
        return carry

    lax.fori_loop(0, _CPW // _NBUF, outer, 0)

    last = base + _CPW - 1
    for d in range(_NBUF - 2, -1, -1):
        p = (_CPW - 1 - d) % _NBUF
        pltpu.make_async_copy(a_hbm.at[gbufs[p]], rowss[p], gsems[p]).wait()
        pltpu.make_async_copy(rowss[p], g_hbm.at[last - d], osems[p]).start()
    for d in range(_NBUF - 1, -1, -1):
        p = (_CPW - 1 - d) % _NBUF
        pltpu.make_async_copy(rowss[p], g_hbm.at[last - d], osems[p]).wait()


def _select_gather(spos, aux_flat, a_pad):
    mesh = plsc.VectorSubcoreMesh(core_axis_name="c", subcore_axis_name="s")
    fn = functools.partial(
        pl.kernel,
        out_type=jax.ShapeDtypeStruct((_B * _S, _K, 128), jnp.float32),
        mesh=mesh,
        compiler_params=pltpu.CompilerParams(needs_layout_passes=False),
        scratch_types=(
            [pltpu.VMEM((16, 128), jnp.int32)] * _NBUF
            + [pltpu.VMEM((_K,), jnp.int32)] * _NBUF
            + [pltpu.VMEM((_K, 128), jnp.float32)] * _NBUF
            + [pltpu.VMEM((_CPW * 128,), jnp.int32),
               pltpu.VMEM((80,), jnp.int32)]
            + [pltpu.SemaphoreType.DMA] * (3 * _NBUF)
        ),
    )(_sc_body)
    return fn(spos, aux_flat, a_pad)


def _pad_w(w, out_lanes=128):
    o, c = w.shape
    wt = jnp.zeros((128, out_lanes), jnp.float32)
    return wt.at[:c, :o].set(w.T)


def kernel(xyz, points, W0, b0, g0, be0, W1, b1, g1, be1, W2, b2, g2, be2):
    xs = xyz[..., 0]
    ys = xyz[..., 1]
    zs = xyz[..., 2]
    cx, cy, cz = _run_fps(xs, ys, zs)
    new_xyz = jnp.stack([cx, cy, cz], axis=-1)

    w0p_pad = _pad_w(W0[:, :64])
    w0x_pad = jnp.zeros((8, 128), jnp.float32).at[:3, :64].set(W0[:, 64:].T)
    b0_pad = jnp.zeros((1, 128), jnp.float32).at[0, :64].set(b0)
    pts_flat = points.reshape(_B * _N, 64)
    pts_pad = jnp.concatenate(
        [pts_flat, jnp.zeros((_B * _N, 64), jnp.float32)], axis=1)
    xyz_flat8 = jnp.concatenate(
        [xyz.reshape(_B * _N, 3), jnp.zeros((_B * _N, 5), jnp.float32)], axis=1)
    nxyz_flat8 = jnp.concatenate(
        [new_xyz.reshape(_B * _S, 3), jnp.zeros((_B * _S, 5), jnp.float32)],
        axis=1)
    a_pad, c_pad = _run_dense(pts_pad, xyz_flat8, nxyz_flat8,
                              w0p_pad, w0x_pad, b0_pad)

    tri = jnp.asarray(np.triu(np.ones((128, 128), np.float32), 1))
    spos, aux, st1 = _run_select(new_xyz, xyz, a_pad, c_pad, tri)
    g = _select_gather(spos, aux.reshape(-1), a_pad)

    bg1 = jnp.zeros((4, 128), jnp.float32)
    bg1 = bg1.at[0, :64].set(b1).at[2, :64].set(g0).at[3, :64].set(be0)
    bg2 = jnp.zeros((8, 128), jnp.float32)
    bg2 = bg2.at[0, :].set(b2).at[2, :64].set(g1).at[3, :64].set(be1)
    bg2 = bg2.at[6, :].set(g2).at[7, :].set(be2)
    new_points = _run_mlp(g, c_pad, st1, _pad_w(W1), _pad_w(W2), bg1, bg2)

    return new_xyz, new_points.reshape(_B, _S, 128)


# speedup vs baseline: 1.1410x; 1.1410x over previous
import functools
import math

import jax
import jax.numpy as jnp
import numpy as np
from jax import lax
from jax.experimental import pallas as pl
from jax.experimental.pallas import tpu as pltpu
from jax.experimental.pallas import tpu_sc as plsc

_INTERPRET = False

_B = 8
_N = 2048
_S = 512
_K = 64
_EPS = 1e-5
_R2 = np.float32(0.2 ** 2)
_M = _B * _S * _K


def _fps_body(x_ref, y_ref, z_ref, cx_ref, cy_ref, cz_ref):
    x = x_ref[...]
    y = y_ref[...]
    z = z_ref[...]
    iota = lax.broadcasted_iota(jnp.int32, (_B, _N), 1)
    iota_s = lax.broadcasted_iota(jnp.int32, (_B, _S), 1)

    def body(i, carry):
        dist, far, cxs, cys, czs = carry
        m = iota == far
        cx = jnp.sum(jnp.where(m, x, 0.0), axis=1, keepdims=True)
        cy = jnp.sum(jnp.where(m, y, 0.0), axis=1, keepdims=True)
        cz = jnp.sum(jnp.where(m, z, 0.0), axis=1, keepdims=True)
        sm = iota_s == i
        cxs = jnp.where(sm, cx, cxs)
        cys = jnp.where(sm, cy, cys)
        czs = jnp.where(sm, cz, czs)
        dx = x - cx
        dy = y - cy
        dz = z - cz
        d = (dx * dx + dy * dy) + dz * dz
        dist = jnp.minimum(dist, d)
        mx = jnp.max(dist, axis=1, keepdims=True)
        far = jnp.min(jnp.where(dist == mx, iota, _N), axis=1, keepdims=True)
        return dist, far, cxs, cys, czs

    dist0 = jnp.full((_B, _N), 1e10, jnp.float32)
    far0 = jnp.zeros((_B, 1), jnp.int32)
    zs0 = jnp.zeros((_B, _S), jnp.float32)
    _, _, cxs, cys, czs = lax.fori_loop(0, _S, body,
                                        (dist0, far0, zs0, zs0, zs0))
    cx_ref[...] = cxs
    cy_ref[...] = cys
    cz_ref[...] = czs


def _run_fps(xs, ys, zs):
    return pl.pallas_call(
        _fps_body,
        out_shape=(
            jax.ShapeDtypeStruct((_B, _S), jnp.float32),
            jax.ShapeDtypeStruct((_B, _S), jnp.float32),
            jax.ShapeDtypeStruct((_B, _S), jnp.float32),
        ),
        interpret=_INTERPRET,
    )(xs, ys, zs)


def _dense_body(pts_ref, xyzf_ref, nxyzf_ref, w0p_ref, w0x_ref, b0_ref,
                a_ref, c_ref):
    a = lax.dot_general(pts_ref[...], w0p_ref[...], (((1,), (0,)), ((), ())),
                        precision=lax.Precision.HIGHEST,
                        preferred_element_type=jnp.float32)
    a = a + lax.dot_general(xyzf_ref[...], w0x_ref[...],
                            (((1,), (0,)), ((), ())),
                            precision=lax.Precision.HIGHEST,
                            preferred_element_type=jnp.float32)
    a_ref[...] = a + b0_ref[...]
    c_ref[...] = lax.dot_general(nxyzf_ref[...], w0x_ref[...],
                                 (((1,), (0,)), ((), ())),
                                 precision=lax.Precision.HIGHEST,
                                 preferred_element_type=jnp.float32)


def _run_dense(points_flat, xyz_flat, nxyz_flat, w0p_pad, w0x_pad, b0_pad,
               tiles=16):
    ra = _B * _N // tiles
    rc = _B * _S // tiles
    return pl.pallas_call(
        _dense_body,
        grid=(tiles,),
        in_specs=[
            pl.BlockSpec((ra, 128), lambda t: (t, 0)),
            pl.BlockSpec((ra, 8), lambda t: (t, 0)),
            pl.BlockSpec((rc, 8), lambda t: (t, 0)),
            pl.BlockSpec((128, 128), lambda t: (0, 0)),
            pl.BlockSpec((8, 128), lambda t: (0, 0)),
            pl.BlockSpec((1, 128), lambda t: (0, 0)),
        ],
        out_specs=(
            pl.BlockSpec((ra, 128), lambda t: (t, 0)),
            pl.BlockSpec((rc, 128), lambda t: (t, 0)),
        ),
        out_shape=(
            jax.ShapeDtypeStruct((_B * _N, 128), jnp.float32),
            jax.ShapeDtypeStruct((_B * _S, 128), jnp.float32),
        ),
        interpret=_INTERPRET,
    )(points_flat, xyz_flat, nxyz_flat, w0p_pad, w0x_pad, b0_pad)


def _sel_body(nxyz_ref, xyz_ref, a_ref, c_ref, tri_ref, spos_ref, aux_ref,
              st1_ref, acc):
    t = pl.program_id(0)
    a = nxyz_ref[0]
    b = xyz_ref[0]
    mm = lax.dot_general(a, b, (((1,), (1,)), ((), ())),
                         preferred_element_type=jnp.float32)
    d = -2.0 * mm
    d = d + jnp.sum(a * a, -1)[:, None]
    d = d + jnp.sum(b * b, -1)[None, :]
    inr = jnp.logical_not(d > _R2)
    maskf = jnp.where(inr, 1.0, 0.0).astype(jnp.float32)
    tri = tri_ref[...]
    lane = lax.broadcasted_iota(jnp.int32, (_S, 128), 1)
    lanef = lane.astype(jnp.float32)

    base = jnp.zeros((_S, 1), jnp.float32)
    first = jnp.full((_S, 1), jnp.float32(_N * 2))
    poss = []
    for j in range(16):
        mj = maskf[:, j * 128:(j + 1) * 128]
        pos_j = lax.dot_general(mj, tri, (((1,), (0,)), ((), ())),
                                preferred_element_type=jnp.float32) + base
        poss.append(pos_j)
        base = base + jnp.sum(mj, axis=1, keepdims=True)
        nval = lanef + jnp.float32(j * 128)
        first = jnp.minimum(
            first, jnp.min(jnp.where(mj > 0.0, nval, jnp.float32(_N * 2)),
                           axis=1, keepdims=True))
    cnt = base
    selm_parts = []
    trash = jnp.float32(_K) + (lane & 15).astype(jnp.float32)
    for j in range(16):
        mj = maskf[:, j * 128:(j + 1) * 128]
        validj = (mj > 0.0) & (poss[j] < jnp.float32(_K))
        spos_ref[:, j, :] = jnp.where(validj, poss[j], trash).astype(jnp.int32)
        selm_parts.append(jnp.where(validj, 1.0, 0.0).astype(jnp.float32))
    selm = jnp.concatenate(selm_parts, axis=1)

    padw = jnp.maximum(jnp.float32(_K) - cnt, 0.0)
    iota_n = lax.broadcasted_iota(jnp.int32, (_S, _N), 1).astype(jnp.float32)
    padoh = jnp.where(iota_n == first, padw, 0.0)
    selmtot = selm + padoh

    aux = (jnp.where(lane == 0, cnt.astype(jnp.int32), 0)
           + jnp.where(lane == 1, first.astype(jnp.int32), 0))
    aux_ref[...] = aux

    av = a_ref[...]
    cv = c_ref[...]
    h = lax.dot_general(selmtot, av, (((1,), (0,)), ((), ())),
                        preferred_element_type=jnp.float32)
    co = jnp.sum(selmtot, axis=0, keepdims=True)
    coa2 = lax.dot_general(co, av * av, (((1,), (0,)), ((), ())),
                           preferred_element_type=jnp.float32)[0]
    s1 = jnp.sum(h, axis=0) - jnp.float32(_K) * jnp.sum(cv, axis=0)
    s2 = (coa2 - 2.0 * jnp.sum(cv * h, axis=0)
          + jnp.float32(_K) * jnp.sum(cv * cv, axis=0))
    st = jnp.stack([s1, s2])

    @pl.when(t == 0)
    def _init():
        acc[...] = st

    @pl.when(t > 0)
    def _accum():
        acc[...] = acc[...] + st

    @pl.when(t == pl.num_programs(0) - 1)
    def _emit():
        st1_ref[...] = acc[...]


def _run_select(new_xyz, xyz, a_pad, c_pad, tri):
    return pl.pallas_call(
        _sel_body,
        grid=(_B,),
        in_specs=[pl.BlockSpec((1, _S, 3), lambda i: (i, 0, 0)),
                  pl.BlockSpec((1, _N, 3), lambda i: (i, 0, 0)),
                  pl.BlockSpec((_N, 128), lambda i: (i, 0)),
                  pl.BlockSpec((_S, 128), lambda i: (i, 0)),
                  pl.BlockSpec((128, 128), lambda i: (0, 0))],
        out_specs=(
            pl.BlockSpec((_S, 16, 128), lambda i: (i, 0, 0)),
            pl.BlockSpec((_S, 128), lambda i: (i, 0)),
            pl.BlockSpec((2, 128), lambda i: (0, 0)),
        ),
        out_shape=(
            jax.ShapeDtypeStruct((_B * _S, 16, 128), jnp.int32),
            jax.ShapeDtypeStruct((_B * _S, 128), jnp.int32),
            jax.ShapeDtypeStruct((2, 128), jnp.float32),
        ),
        scratch_shapes=[pltpu.VMEM((2, 128), jnp.float32)],
        interpret=_INTERPRET,
    )(new_xyz, xyz, a_pad, c_pad, tri)


def _stats1_body(g_ref, c_ref, o_ref, acc):
    t = pl.program_id(0)
    z = g_ref[...] - c_ref[...][:, None, :]
    s1 = jnp.sum(z, axis=(0, 1))
    s2 = jnp.sum(z * z, axis=(0, 1))
    st = jnp.stack([s1, s2])

    @pl.when(t == 0)
    def _():
        acc[...] = st

    @pl.when(t > 0)
    def _():
        acc[...] = acc[...] + st

    @pl.when(t == pl.num_programs(0) - 1)
    def _():
        o_ref[...] = acc[...]


def _run_stats1(g, c, tiles=32):
    rows = _B * _S // tiles
    return pl.pallas_call(
        _stats1_body,
        grid=(tiles,),
        in_specs=[
            pl.BlockSpec((rows, _K, 128), lambda t: (t, 0, 0)),
            pl.BlockSpec((rows, 128), lambda t: (t, 0)),
        ],
        out_specs=pl.BlockSpec((2, 128), lambda t: (0, 0)),
        out_shape=jax.ShapeDtypeStruct((2, 128), jnp.float32),
        scratch_shapes=[pltpu.VMEM((2, 128), jnp.float32)],
        interpret=_INTERPRET,
    )(g, c)


def _mlp_body(g_ref, c_ref, st1_ref, w1_ref, w2_ref, bg1_ref, bg2_ref,
              out_ref, acc2, acc3, mx_ref):
    p = pl.program_id(0)
    t = pl.program_id(1)
    nt = pl.num_programs(1)
    st = st1_ref[...]
    mean1 = st[0] * (1.0 / _M)
    var1 = st[1] * (1.0 / _M) - mean1 * mean1
    inv1 = bg1_ref[2] / jnp.sqrt(var1 + _EPS)
    z1 = g_ref[...] - c_ref[...][:, None, :]
    x1 = jnp.maximum((z1 - mean1[None, None, :]) * inv1[None, None, :]
                     + bg1_ref[3][None, None, :], 0.0)
    rows = x1.shape[0] * x1.shape[1]
    z2 = lax.dot_general(x1.reshape(rows, 128), w1_ref[...],
                         (((1,), (0,)), ((), ())),
                         precision=lax.Precision.HIGHEST,
                         preferred_element_type=jnp.float32)
    z2 = z2 + bg1_ref[0][None, :]

    @pl.when(p == 0)
    def _phase_stats():
        s1 = jnp.sum(z2, axis=0)
        s2 = jnp.sum(z2 * z2, axis=0)
        st_new = jnp.stack([s1, s2])

        @pl.when(t == 0)
        def _():
            acc2[...] = st_new

        @pl.when(t > 0)
        def _():
            acc2[...] = acc2[...] + st_new

    @pl.when(p == 1)
    def _phase_layer3():
        st2 = acc2[...]
        mean2 = st2[0] * (1.0 / _M)
        var2 = st2[1] * (1.0 / _M) - mean2 * mean2
        inv2 = bg2_ref[2] / jnp.sqrt(var2 + _EPS)
        x2 = jnp.maximum((z2 - mean2[None, :]) * inv2[None, :]
                         + bg2_ref[3][None, :], 0.0)
        z3 = lax.dot_general(x2, w2_ref[...], (((1,), (0,)), ((), ())),
                             precision=lax.Precision.HIGHEST,
                             preferred_element_type=jnp.float32)
        z3 = z3 + bg2_ref[0][None, :]
        s1 = jnp.sum(z3, axis=0)
        s2 = jnp.sum(z3 * z3, axis=0)
        st_new = jnp.stack([s1, s2])

        @pl.when(t == 0)
        def _():
            acc3[...] = st_new

        @pl.when(t > 0)
        def _():
            acc3[...] = acc3[...] + st_new

        srows = z3.shape[0] // _K
        mx = jnp.max(z3.reshape(srows, _K, 128), axis=1)
        mx_ref[pl.ds(t * srows, srows), :] = mx

        @pl.when(t == nt - 1)
        def _():
            stf = acc3[...]
            mean3 = stf[0] * (1.0 / _M)
            var3 = stf[1] * (1.0 / _M) - mean3 * mean3
            inv3 = bg2_ref[6] / jnp.sqrt(var3 + _EPS)
            pooled = mx_ref[...]
            out_ref[...] = jnp.maximum(
                (pooled - mean3[None, :]) * inv3[None, :]
                + bg2_ref[7][None, :], 0.0)


def _run_mlp(g, c, st1, w1_pad, w2_pad, bg1, bg2, tiles=32):
    rows = _B * _S // tiles
    return pl.pallas_call(
        _mlp_body,
        grid=(2, tiles),
        in_specs=[
            pl.BlockSpec((rows, _K, 128), lambda p, t: (t, 0, 0)),
            pl.BlockSpec((rows, 128), lambda p, t: (t, 0)),
            pl.BlockSpec((2, 128), lambda p, t: (0, 0)),
            pl.BlockSpec((128, 128), lambda p, t: (0, 0)),
            pl.BlockSpec((128, 128), lambda p, t: (0, 0)),
            pl.BlockSpec((4, 128), lambda p, t: (0, 0)),
            pl.BlockSpec((8, 128), lambda p, t: (0, 0)),
        ],
        out_specs=pl.BlockSpec((_B * _S, 128), lambda p, t: (0, 0)),
        out_shape=jax.ShapeDtypeStruct((_B * _S, 128), jnp.float32),
        scratch_shapes=[
            pltpu.VMEM((2, 128), jnp.float32),
            pltpu.VMEM((2, 128), jnp.float32),
            pltpu.VMEM((_B * _S, 128), jnp.float32),
        ],
        interpret=_INTERPRET,
    )(g, c, st1, w1_pad, w2_pad, bg1, bg2)


def _layer2_body(g_ref, c_ref, st1_ref, w1_ref, bg_ref, z2_ref, st2_ref, acc):
    t = pl.program_id(0)
    st = st1_ref[...]
    mean = st[0] * (1.0 / _M)
    var = st[1] * (1.0 / _M) - mean * mean
    inv = bg_ref[2] / jnp.sqrt(var + _EPS)
    d = bg_ref[3][None, :] - (c_ref[...] + mean[None, :]) * inv[None, :]
    x1 = jnp.maximum(g_ref[...] * inv[None, None, :] + d[:, None, :], 0.0)
    rows = x1.shape[0] * x1.shape[1]
    z2 = lax.dot_general(x1.reshape(rows, 128), w1_ref[...],
                         (((1,), (0,)), ((), ())),
                         precision=lax.Precision.HIGHEST,
                         preferred_element_type=jnp.float32)
    z2 = z2 + bg_ref[0][None, :]
    z2_ref[...] = z2
    s1 = jnp.sum(z2, axis=0)
    gram = lax.dot_general(z2, z2, (((0,), (0,)), ((), ())),
                           preferred_element_type=jnp.float32)
    eye = (lax.broadcasted_iota(jnp.int32, (128, 128), 0)
           == lax.broadcasted_iota(jnp.int32, (128, 128), 1))
    s2 = jnp.sum(jnp.where(eye, gram, 0.0), axis=0)
    st_new = jnp.stack([s1, s2])

    @pl.when(t == 0)
    def _():
        acc[...] = st_new

    @pl.when(t > 0)
    def _():
        acc[...] = acc[...] + st_new

    @pl.when(t == pl.num_programs(0) - 1)
    def _():
        st2_ref[...] = acc[...]


def _run_layer2(g, c, st1, w1_pad, bg1, tiles=32):
    rows = _B * _S // tiles
    return pl.pallas_call(
        _layer2_body,
        grid=(tiles,),
        in_specs=[
            pl.BlockSpec((rows, _K, 128), lambda t: (t, 0, 0)),
            pl.BlockSpec((rows, 128), lambda t: (t, 0)),
            pl.BlockSpec((2, 128), lambda t: (0, 0)),
            pl.BlockSpec((128, 128), lambda t: (0, 0)),
            pl.BlockSpec((4, 128), lambda t: (0, 0)),
        ],
        out_specs=(
            pl.BlockSpec((rows * _K, 128), lambda t: (t, 0)),
            pl.BlockSpec((2, 128), lambda t: (0, 0)),
        ),
        out_shape=(
            jax.ShapeDtypeStruct((_B * _S * _K, 128), jnp.float32),
            jax.ShapeDtypeStruct((2, 128), jnp.float32),
        ),
        scratch_shapes=[pltpu.VMEM((2, 128), jnp.float32)],
        interpret=_INTERPRET,
    )(g, c, st1, w1_pad, bg1)


def _layer3_body(z2_ref, st2_ref, w2_ref, bg_ref, out_ref, acc, mx_ref):
    t = pl.program_id(0)
    nt = pl.num_programs(0)
    st = st2_ref[...]
    mean = st[0] * (1.0 / _M)
    var = st[1] * (1.0 / _M) - mean * mean
    inv = bg_ref[2] / jnp.sqrt(var + _EPS)
    cst = bg_ref[3] - mean * inv
    x2 = jnp.maximum(z2_ref[...] * inv[None, :] + cst[None, :], 0.0)
    z3 = lax.dot_general(x2, w2_ref[...], (((1,), (0,)), ((), ())),
                         precision=lax.Precision.HIGHEST,
                         preferred_element_type=jnp.float32)
    z3 = z3 + bg_ref[0][None, :]
    s1 = jnp.sum(z3, axis=0)
    gram = lax.dot_general(z3, z3, (((0,), (0,)), ((), ())),
                           preferred_element_type=jnp.float32)
    eye = (lax.broadcasted_iota(jnp.int32, (128, 128), 0)
           == lax.broadcasted_iota(jnp.int32, (128, 128), 1))
    s2 = jnp.sum(jnp.where(eye, gram, 0.0), axis=0)
    st_new = jnp.stack([s1, s2])

    @pl.when(t == 0)
    def _():
        acc[...] = st_new

    @pl.when(t > 0)
    def _():
        acc[...] = acc[...] + st_new

    rows = z3.shape[0] // _K
    mx = jnp.max(z3.reshape(rows, _K, 128), axis=1)
    mx_ref[pl.ds(t * rows, rows), :] = mx

    @pl.when(t == nt - 1)
    def _():
        stf = acc[...]
        mean3 = stf[0] * (1.0 / _M)
        var3 = stf[1] * (1.0 / _M) - mean3 * mean3
        inv3 = bg_ref[6] / jnp.sqrt(var3 + _EPS)
        pooled = mx_ref[...]
        out_ref[...] = jnp.maximum(
            (pooled - mean3[None, :]) * inv3[None, :] + bg_ref[7][None, :], 0.0)


def _run_layer3(z2, st2, w2_pad, bg2, tiles=32):
    rows = _B * _S * _K // tiles
    return pl.pallas_call(
        _layer3_body,
        grid=(tiles,),
        in_specs=[
            pl.BlockSpec((rows, 128), lambda t: (t, 0)),
            pl.BlockSpec((2, 128), lambda t: (0, 0)),
            pl.BlockSpec((128, 128), lambda t: (0, 0)),
            pl.BlockSpec((8, 128), lambda t: (0, 0)),
        ],
        out_specs=pl.BlockSpec((_B * _S, 128), lambda t: (0, 0)),
        out_shape=jax.ShapeDtypeStruct((_B * _S, 128), jnp.float32),
        scratch_shapes=[
            pltpu.VMEM((2, 128), jnp.float32),
            pltpu.VMEM((_B * _S, 128), jnp.float32),
        ],
        interpret=_INTERPRET,
    )(z2, st2, w2_pad, bg2)


_NW = 32
_CPW = (_B * _S) // _NW


_NBUF = 4


def _sc_body(spos_hbm, aux_hbm, a_hbm, g_hbm, *refs):
    drows = refs[0:4]
    gbufs = refs[4:8]
    rowss = refs[8:12]
    aux_v = refs[12]
    selbuf = refs[13]
    dsems = refs[14:18]
    gsems = refs[18:22]
    osems = refs[22:26]
    cid = lax.axis_index("c")
    sid = lax.axis_index("s")
    wid = sid * 2 + cid
    base = wid * _CPW
    iota16 = lax.iota(jnp.int32, 16)

    pltpu.sync_copy(aux_hbm.at[pl.ds(base * 128, _CPW * 128)], aux_v)

    def select(drow, gbuf, i, s):
        for r in range(16):
            for l in range(8):
                sp16 = drow[r, pl.ds(l * 16, 16)]
                vals = iota16 + (r * 128 + l * 16)
                plsc.store_scatter(selbuf, [sp16], vals)
        v16 = aux_v[pl.ds(i * 128, 16)]
        cnt = lax.squeeze(lax.slice(v16, (0,), (1,)), (0,))
        first = lax.squeeze(lax.slice(v16, (1,), (2,)), (0,))
        off = lax.shift_right_logical(s, 9) * _N
        for q in range(4):
            cur = selbuf[pl.ds(q * 16, 16)]
            sel = jnp.where((iota16 + q * 16) < cnt, cur, first)
            gbuf[pl.ds(q * 16, 16)] = sel + off

    for p in range(_NBUF - 1):
        pltpu.make_async_copy(spos_hbm.at[base + p], drows[p],
                              dsems[p]).start()

    def outer(gi, carry):
        for par in range(_NBUF):
            i = gi * _NBUF + par
            s = base + i
            pltpu.make_async_copy(spos_hbm.at[s], drows[par],
                                  dsems[par]).wait()

            @pl.when(i + _NBUF - 1 < _CPW)
            def _start_next_dist():
                nx = (par + _NBUF - 1) % _NBUF
                pltpu.make_async_copy(spos_hbm.at[s + _NBUF - 1],
                                      drows[nx], dsems[nx]).start()

            select(drows[par], gbufs[par], i, s)

            @pl.when(i >= _NBUF)
            def _wait_out_copy():
                pltpu.make_async_copy(rowss[par], g_hbm.at[s - _NBUF],
                                      osems[par]).wait()

            pltpu.make_async_copy(a_hbm.at[gbufs[par]], rowss[par],
                                  gsems[par]).start()

            @pl.when(i >= _NBUF - 1)
            def _drain_old_gather():
                pv = (par + 1) % _NBUF
                pltpu.make_async_copy(a_hbm.at[gbufs[pv]],
                                      rowss[pv], gsems[pv]).wait()
                pltpu.make_async_copy(rowss[pv], g_hbm.at[s - (_NBUF - 1)],
                                      osems[pv]).start()
        return carry

    lax.fori_loop(0, _CPW // _NBUF, outer, 0)

    last = base + _CPW - 1
    for d in range(_NBUF - 2, -1, -1):
        p = (_CPW - 1 - d) % _NBUF
        pltpu.make_async_copy(a_hbm.at[gbufs[p]], rowss[p], gsems[p]).wait()
        pltpu.make_async_copy(rowss[p], g_hbm.at[last - d], osems[p]).start()
    for d in range(_NBUF - 1, -1, -1):
        p = (_CPW - 1 - d) % _NBUF
        pltpu.make_async_copy(rowss[p], g_hbm.at[last - d], osems[p]).wait()


def _select_gather(spos, aux_flat, a_pad):
    mesh = plsc.VectorSubcoreMesh(core_axis_name="c", subcore_axis_name="s")
    fn = functools.partial(
        pl.kernel,
        out_type=jax.ShapeDtypeStruct((_B * _S, _K, 128), jnp.float32),
        mesh=mesh,
        compiler_params=pltpu.CompilerParams(needs_layout_passes=False),
        scratch_types=(
            [pltpu.VMEM((16, 128), jnp.int32)] * _NBUF
            + [pltpu.VMEM((_K,), jnp.int32)] * _NBUF
            + [pltpu.VMEM((_K, 128), jnp.float32)] * _NBUF
            + [pltpu.VMEM((_CPW * 128,), jnp.int32),
               pltpu.VMEM((80,), jnp.int32)]
            + [pltpu.SemaphoreType.DMA] * (3 * _NBUF)
        ),
    )(_sc_body)
    return fn(spos, aux_flat, a_pad)


def _pad_w(w, out_lanes=128):
    o, c = w.shape
    wt = jnp.zeros((128, out_lanes), jnp.float32)
    return wt.at[:c, :o].set(w.T)


def kernel(xyz, points, W0, b0, g0, be0, W1, b1, g1, be1, W2, b2, g2, be2):
    xs = xyz[..., 0]
    ys = xyz[..., 1]
    zs = xyz[..., 2]
    cx, cy, cz = _run_fps(xs, ys, zs)
    new_xyz = jnp.stack([cx, cy, cz], axis=-1)

    w0p_pad = _pad_w(W0[:, :64])
    w0x_pad = jnp.zeros((8, 128), jnp.float32).at[:3, :64].set(W0[:, 64:].T)
    b0_pad = jnp.zeros((1, 128), jnp.float32).at[0, :64].set(b0)
    pts_flat = points.reshape(_B * _N, 64)
    pts_pad = jnp.concatenate(
        [pts_flat, jnp.zeros((_B * _N, 64), jnp.float32)], axis=1)
    xyz_flat8 = jnp.concatenate(
        [xyz.reshape(_B * _N, 3), jnp.zeros((_B * _N, 5), jnp.float32)], axis=1)
    nxyz_flat8 = jnp.concatenate(
        [new_xyz.reshape(_B * _S, 3), jnp.zeros((_B * _S, 5), jnp.float32)],
        axis=1)
    a_pad, c_pad = _run_dense(pts_pad, xyz_flat8, nxyz_flat8,
                              w0p_pad, w0x_pad, b0_pad)

    tri = jnp.asarray(np.triu(np.ones((128, 128), np.float32), 1))
    spos, aux, st1 = _run_select(new_xyz, xyz, a_pad, c_pad, tri)
    g = _select_gather(spos, aux.reshape(-1), a_pad)

    bg1 = jnp.zeros((4, 128), jnp.float32)
    bg1 = bg1.at[0, :64].set(b1).at[2, :64].set(g0).at[3, :64].set(be0)
    bg2 = jnp.zeros((8, 128), jnp.float32)
    bg2 = bg2.at[0, :].set(b2).at[2, :64].set(g1).at[3, :64].set(be1)
    bg2 = bg2.at[6, :].set(g2).at[7, :].set(be2)
    z2, st2 = _run_layer2(g, c_pad, st1, _pad_w(W1), bg1)
    new_points = _run_layer3(z2, st2, _pad_w(W2), bg2)

    return new_xyz, new_points.reshape(_B, _S, 128)


# speedup vs baseline: 1.2121x; 1.0624x over previous
import functools
import math

import jax
import jax.numpy as jnp
import numpy as np
from jax import lax
from jax.experimental import pallas as pl
from jax.experimental.pallas import tpu as pltpu
from jax.experimental.pallas import tpu_sc as plsc

_INTERPRET = False

_B = 8
_N = 2048
_S = 512
_K = 64
_EPS = 1e-5
_R2 = np.float32(0.2 ** 2)
_M = _B * _S * _K


def _fps_body(x_ref, y_ref, z_ref, cx_ref, cy_ref, cz_ref):
    x = x_ref[...]
    y = y_ref[...]
    z = z_ref[...]
    iota = lax.broadcasted_iota(jnp.int32, (_B, _N), 1)
    iota_s = lax.broadcasted_iota(jnp.int32, (_B, _S), 1)

    def body(i, carry):
        dist, far, cxs, cys, czs = carry
        m = iota == far
        cx = jnp.sum(jnp.where(m, x, 0.0), axis=1, keepdims=True)
        cy = jnp.sum(jnp.where(m, y, 0.0), axis=1, keepdims=True)
        cz = jnp.sum(jnp.where(m, z, 0.0), axis=1, keepdims=True)
        sm = iota_s == i
        cxs = jnp.where(sm, cx, cxs)
        cys = jnp.where(sm, cy, cys)
        czs = jnp.where(sm, cz, czs)
        dx = x - cx
        dy = y - cy
        dz = z - cz
        d = (dx * dx + dy * dy) + dz * dz
        dist = jnp.minimum(dist, d)
        mx = jnp.max(dist, axis=1, keepdims=True)
        far = jnp.min(jnp.where(dist == mx, iota, _N), axis=1, keepdims=True)
        return dist, far, cxs, cys, czs

    dist0 = jnp.full((_B, _N), 1e10, jnp.float32)
    far0 = jnp.zeros((_B, 1), jnp.int32)
    zs0 = jnp.zeros((_B, _S), jnp.float32)
    _, _, cxs, cys, czs = lax.fori_loop(0, _S, body,
                                        (dist0, far0, zs0, zs0, zs0))
    cx_ref[...] = cxs
    cy_ref[...] = cys
    cz_ref[...] = czs


def _run_fps(xs, ys, zs):
    return pl.pallas_call(
        _fps_body,
        out_shape=(
            jax.ShapeDtypeStruct((_B, _S), jnp.float32),
            jax.ShapeDtypeStruct((_B, _S), jnp.float32),
            jax.ShapeDtypeStruct((_B, _S), jnp.float32),
        ),
        interpret=_INTERPRET,
    )(xs, ys, zs)


def _dense_body(pts_ref, xyzf_ref, nxyzf_ref, w0p_ref, w0x_ref, b0_ref,
                a_ref, c_ref):
    a = lax.dot_general(pts_ref[...], w0p_ref[...], (((1,), (0,)), ((), ())),
                        precision=lax.Precision.HIGHEST,
                        preferred_element_type=jnp.float32)
    a = a + lax.dot_general(xyzf_ref[...], w0x_ref[...],
                            (((1,), (0,)), ((), ())),
                            precision=lax.Precision.HIGHEST,
                            preferred_element_type=jnp.float32)
    a_ref[...] = a + b0_ref[...]
    c_ref[...] = lax.dot_general(nxyzf_ref[...], w0x_ref[...],
                                 (((1,), (0,)), ((), ())),
                                 precision=lax.Precision.HIGHEST,
                                 preferred_element_type=jnp.float32)


def _run_dense(points_flat, xyz_flat, nxyz_flat, w0p_pad, w0x_pad, b0_pad,
               tiles=16):
    ra = _B * _N // tiles
    rc = _B * _S // tiles
    return pl.pallas_call(
        _dense_body,
        grid=(tiles,),
        in_specs=[
            pl.BlockSpec((ra, 128), lambda t: (t, 0)),
            pl.BlockSpec((ra, 8), lambda t: (t, 0)),
            pl.BlockSpec((rc, 8), lambda t: (t, 0)),
            pl.BlockSpec((128, 128), lambda t: (0, 0)),
            pl.BlockSpec((8, 128), lambda t: (0, 0)),
            pl.BlockSpec((1, 128), lambda t: (0, 0)),
        ],
        out_specs=(
            pl.BlockSpec((ra, 128), lambda t: (t, 0)),
            pl.BlockSpec((rc, 128), lambda t: (t, 0)),
        ),
        out_shape=(
            jax.ShapeDtypeStruct((_B * _N, 128), jnp.float32),
            jax.ShapeDtypeStruct((_B * _S, 128), jnp.float32),
        ),
        interpret=_INTERPRET,
    )(points_flat, xyz_flat, nxyz_flat, w0p_pad, w0x_pad, b0_pad)


def _sel_body(nxyz_ref, xyz_ref, a_ref, c_ref, tri_ref, spos_ref, aux_ref,
              st1_ref, acc):
    t = pl.program_id(0)
    a = nxyz_ref[0]
    b = xyz_ref[0]
    mm = lax.dot_general(a, b, (((1,), (1,)), ((), ())),
                         preferred_element_type=jnp.float32)
    d = -2.0 * mm
    d = d + jnp.sum(a * a, -1)[:, None]
    d = d + jnp.sum(b * b, -1)[None, :]
    inr = jnp.logical_not(d > _R2)
    maskf = jnp.where(inr, 1.0, 0.0).astype(jnp.float32)
    tri = tri_ref[...]
    lane = lax.broadcasted_iota(jnp.int32, (_S, 128), 1)
    lanef = lane.astype(jnp.float32)

    base = jnp.zeros((_S, 1), jnp.float32)
    first = jnp.full((_S, 1), jnp.float32(_N * 2))
    poss = []
    for j in range(16):
        mj = maskf[:, j * 128:(j + 1) * 128]
        pos_j = lax.dot_general(mj, tri, (((1,), (0,)), ((), ())),
                                preferred_element_type=jnp.float32) + base
        poss.append(pos_j)
        base = base + jnp.sum(mj, axis=1, keepdims=True)
        nval = lanef + jnp.float32(j * 128)
        first = jnp.minimum(
            first, jnp.min(jnp.where(mj > 0.0, nval, jnp.float32(_N * 2)),
                           axis=1, keepdims=True))
    cnt = base
    selm_parts = []
    trash = jnp.float32(_K) + (lane & 15).astype(jnp.float32)
    for j in range(16):
        mj = maskf[:, j * 128:(j + 1) * 128]
        validj = (mj > 0.0) & (poss[j] < jnp.float32(_K))
        spos_ref[:, j, :] = jnp.where(validj, poss[j], trash).astype(jnp.int32)
        selm_parts.append(jnp.where(validj, 1.0, 0.0).astype(jnp.float32))
    selm = jnp.concatenate(selm_parts, axis=1)

    padw = jnp.maximum(jnp.float32(_K) - cnt, 0.0)
    iota_n = lax.broadcasted_iota(jnp.int32, (_S, _N), 1).astype(jnp.float32)
    padoh = jnp.where(iota_n == first, padw, 0.0)
    selmtot = selm + padoh

    aux = (jnp.where(lane == 0, cnt.astype(jnp.int32), 0)
           + jnp.where(lane == 1, first.astype(jnp.int32), 0))
    aux_ref[...] = aux

    av = a_ref[...]
    cv = c_ref[...]
    h = lax.dot_general(selmtot, av, (((1,), (0,)), ((), ())),
                        preferred_element_type=jnp.float32)
    co = jnp.sum(selmtot, axis=0, keepdims=True)
    coa2 = lax.dot_general(co, av * av, (((1,), (0,)), ((), ())),
                           preferred_element_type=jnp.float32)[0]
    s1 = jnp.sum(h, axis=0) - jnp.float32(_K) * jnp.sum(cv, axis=0)
    s2 = (coa2 - 2.0 * jnp.sum(cv * h, axis=0)
          + jnp.float32(_K) * jnp.sum(cv * cv, axis=0))
    st = jnp.stack([s1, s2])

    @pl.when(t == 0)
    def _init():
        acc[...] = st

    @pl.when(t > 0)
    def _accum():
        acc[...] = acc[...] + st

    @pl.when(t == pl.num_programs(0) - 1)
    def _emit():
        st1_ref[...] = acc[...]


def _run_select(new_xyz, xyz, a_pad, c_pad, tri):
    return pl.pallas_call(
        _sel_body,
        grid=(_B,),
        in_specs=[pl.BlockSpec((1, _S, 3), lambda i: (i, 0, 0)),
                  pl.BlockSpec((1, _N, 3), lambda i: (i, 0, 0)),
                  pl.BlockSpec((_N, 128), lambda i: (i, 0)),
                  pl.BlockSpec((_S, 128), lambda i: (i, 0)),
                  pl.BlockSpec((128, 128), lambda i: (0, 0))],
        out_specs=(
            pl.BlockSpec((_S, 16, 128), lambda i: (i, 0, 0)),
            pl.BlockSpec((_S, 128), lambda i: (i, 0)),
            pl.BlockSpec((2, 128), lambda i: (0, 0)),
        ),
        out_shape=(
            jax.ShapeDtypeStruct((_B * _S, 16, 128), jnp.int32),
            jax.ShapeDtypeStruct((_B * _S, 128), jnp.int32),
            jax.ShapeDtypeStruct((2, 128), jnp.float32),
        ),
        scratch_shapes=[pltpu.VMEM((2, 128), jnp.float32)],
        interpret=_INTERPRET,
    )(new_xyz, xyz, a_pad, c_pad, tri)


def _stats1_body(g_ref, c_ref, o_ref, acc):
    t = pl.program_id(0)
    z = g_ref[...] - c_ref[...][:, None, :]
    s1 = jnp.sum(z, axis=(0, 1))
    s2 = jnp.sum(z * z, axis=(0, 1))
    st = jnp.stack([s1, s2])

    @pl.when(t == 0)
    def _():
        acc[...] = st

    @pl.when(t > 0)
    def _():
        acc[...] = acc[...] + st

    @pl.when(t == pl.num_programs(0) - 1)
    def _():
        o_ref[...] = acc[...]


def _run_stats1(g, c, tiles=32):
    rows = _B * _S // tiles
    return pl.pallas_call(
        _stats1_body,
        grid=(tiles,),
        in_specs=[
            pl.BlockSpec((rows, _K, 128), lambda t: (t, 0, 0)),
            pl.BlockSpec((rows, 128), lambda t: (t, 0)),
        ],
        out_specs=pl.BlockSpec((2, 128), lambda t: (0, 0)),
        out_shape=jax.ShapeDtypeStruct((2, 128), jnp.float32),
        scratch_shapes=[pltpu.VMEM((2, 128), jnp.float32)],
        interpret=_INTERPRET,
    )(g, c)


def _mlp_body(g_ref, c_ref, st1_ref, w1_ref, w2_ref, bg1_ref, bg2_ref,
              out_ref, acc2, acc3, mx_ref):
    p = pl.program_id(0)
    t = pl.program_id(1)
    nt = pl.num_programs(1)
    st = st1_ref[...]
    mean1 = st[0] * (1.0 / _M)
    var1 = st[1] * (1.0 / _M) - mean1 * mean1
    inv1 = bg1_ref[2] / jnp.sqrt(var1 + _EPS)
    z1 = g_ref[...] - c_ref[...][:, None, :]
    x1 = jnp.maximum((z1 - mean1[None, None, :]) * inv1[None, None, :]
                     + bg1_ref[3][None, None, :], 0.0)
    rows = x1.shape[0] * x1.shape[1]
    z2 = lax.dot_general(x1.reshape(rows, 128), w1_ref[...],
                         (((1,), (0,)), ((), ())),
                         precision=lax.Precision.HIGHEST,
                         preferred_element_type=jnp.float32)
    z2 = z2 + bg1_ref[0][None, :]

    @pl.when(p == 0)
    def _phase_stats():
        s1 = jnp.sum(z2, axis=0)
        s2 = jnp.sum(z2 * z2, axis=0)
        st_new = jnp.stack([s1, s2])

        @pl.when(t == 0)
        def _():
            acc2[...] = st_new

        @pl.when(t > 0)
        def _():
            acc2[...] = acc2[...] + st_new

    @pl.when(p == 1)
    def _phase_layer3():
        st2 = acc2[...]
        mean2 = st2[0] * (1.0 / _M)
        var2 = st2[1] * (1.0 / _M) - mean2 * mean2
        inv2 = bg2_ref[2] / jnp.sqrt(var2 + _EPS)
        x2 = jnp.maximum((z2 - mean2[None, :]) * inv2[None, :]
                         + bg2_ref[3][None, :], 0.0)
        z3 = lax.dot_general(x2, w2_ref[...], (((1,), (0,)), ((), ())),
                             precision=lax.Precision.HIGHEST,
                             preferred_element_type=jnp.float32)
        z3 = z3 + bg2_ref[0][None, :]
        s1 = jnp.sum(z3, axis=0)
        s2 = jnp.sum(z3 * z3, axis=0)
        st_new = jnp.stack([s1, s2])

        @pl.when(t == 0)
        def _():
            acc3[...] = st_new

        @pl.when(t > 0)
        def _():
            acc3[...] = acc3[...] + st_new

        srows = z3.shape[0] // _K
        mx = jnp.max(z3.reshape(srows, _K, 128), axis=1)
        mx_ref[pl.ds(t * srows, srows), :] = mx

        @pl.when(t == nt - 1)
        def _():
            stf = acc3[...]
            mean3 = stf[0] * (1.0 / _M)
            var3 = stf[1] * (1.0 / _M) - mean3 * mean3
            inv3 = bg2_ref[6] / jnp.sqrt(var3 + _EPS)
            pooled = mx_ref[...]
            out_ref[...] = jnp.maximum(
                (pooled - mean3[None, :]) * inv3[None, :]
                + bg2_ref[7][None, :], 0.0)


def _run_mlp(g, c, st1, w1_pad, w2_pad, bg1, bg2, tiles=32):
    rows = _B * _S // tiles
    return pl.pallas_call(
        _mlp_body,
        grid=(2, tiles),
        in_specs=[
            pl.BlockSpec((rows, _K, 128), lambda p, t: (t, 0, 0)),
            pl.BlockSpec((rows, 128), lambda p, t: (t, 0)),
            pl.BlockSpec((2, 128), lambda p, t: (0, 0)),
            pl.BlockSpec((128, 128), lambda p, t: (0, 0)),
            pl.BlockSpec((128, 128), lambda p, t: (0, 0)),
            pl.BlockSpec((4, 128), lambda p, t: (0, 0)),
            pl.BlockSpec((8, 128), lambda p, t: (0, 0)),
        ],
        out_specs=pl.BlockSpec((_B * _S, 128), lambda p, t: (0, 0)),
        out_shape=jax.ShapeDtypeStruct((_B * _S, 128), jnp.float32),
        scratch_shapes=[
            pltpu.VMEM((2, 128), jnp.float32),
            pltpu.VMEM((2, 128), jnp.float32),
            pltpu.VMEM((_B * _S, 128), jnp.float32),
        ],
        interpret=_INTERPRET,
    )(g, c, st1, w1_pad, w2_pad, bg1, bg2)


def _layer2_body(g_ref, c_ref, st1_ref, w1_ref, bg_ref, z2_ref, st2_ref, acc):
    t = pl.program_id(0)
    st = st1_ref[...]
    mean = st[0] * (1.0 / _M)
    var = st[1] * (1.0 / _M) - mean * mean
    inv = bg_ref[2] / jnp.sqrt(var + _EPS)
    d = bg_ref[3][None, :] - (c_ref[...] + mean[None, :]) * inv[None, :]
    x1 = jnp.maximum(g_ref[...] * inv[None, None, :] + d[:, None, :], 0.0)
    rows = x1.shape[0] * x1.shape[1]
    z2 = lax.dot_general(x1.reshape(rows, 128), w1_ref[...],
                         (((1,), (0,)), ((), ())),
                         precision=lax.Precision.HIGHEST,
                         preferred_element_type=jnp.float32)
    z2 = z2 + bg_ref[0][None, :]
    z2_ref[...] = z2
    s1 = jnp.sum(z2, axis=0)
    s2 = jnp.sum(z2 * z2, axis=0)
    st_new = jnp.stack([s1, s2])

    @pl.when(t == 0)
    def _():
        acc[...] = st_new

    @pl.when(t > 0)
    def _():
        acc[...] = acc[...] + st_new

    @pl.when(t == pl.num_programs(0) - 1)
    def _():
        st2_ref[...] = acc[...]


def _run_layer2(g, c, st1, w1_pad, bg1, tiles=32):
    rows = _B * _S // tiles
    return pl.pallas_call(
        _layer2_body,
        grid=(tiles,),
        in_specs=[
            pl.BlockSpec((rows, _K, 128), lambda t: (t, 0, 0)),
            pl.BlockSpec((rows, 128), lambda t: (t, 0)),
            pl.BlockSpec((2, 128), lambda t: (0, 0)),
            pl.BlockSpec((128, 128), lambda t: (0, 0)),
            pl.BlockSpec((4, 128), lambda t: (0, 0)),
        ],
        out_specs=(
            pl.BlockSpec((rows * _K, 128), lambda t: (t, 0)),
            pl.BlockSpec((2, 128), lambda t: (0, 0)),
        ),
        out_shape=(
            jax.ShapeDtypeStruct((_B * _S * _K, 128), jnp.float32),
            jax.ShapeDtypeStruct((2, 128), jnp.float32),
        ),
        scratch_shapes=[pltpu.VMEM((2, 128), jnp.float32)],
        interpret=_INTERPRET,
    )(g, c, st1, w1_pad, bg1)


def _layer3_body(z2_ref, st2_ref, w2_ref, bg_ref, out_ref, acc, mx_ref):
    t = pl.program_id(0)
    nt = pl.num_programs(0)
    st = st2_ref[...]
    mean = st[0] * (1.0 / _M)
    var = st[1] * (1.0 / _M) - mean * mean
    inv = bg_ref[2] / jnp.sqrt(var + _EPS)
    cst = bg_ref[3] - mean * inv
    x2 = jnp.maximum(z2_ref[...] * inv[None, :] + cst[None, :], 0.0)
    z3 = lax.dot_general(x2, w2_ref[...], (((1,), (0,)), ((), ())),
                         precision=lax.Precision.HIGHEST,
                         preferred_element_type=jnp.float32)
    z3 = z3 + bg_ref[0][None, :]
    s1 = jnp.sum(z3, axis=0)
    s2 = jnp.sum(z3 * z3, axis=0)
    st_new = jnp.stack([s1, s2])

    @pl.when(t == 0)
    def _():
        acc[...] = st_new

    @pl.when(t > 0)
    def _():
        acc[...] = acc[...] + st_new

    rows = z3.shape[0] // _K
    mx = jnp.max(z3.reshape(rows, _K, 128), axis=1)
    mx_ref[pl.ds(t * rows, rows), :] = mx

    @pl.when(t == nt - 1)
    def _():
        stf = acc[...]
        mean3 = stf[0] * (1.0 / _M)
        var3 = stf[1] * (1.0 / _M) - mean3 * mean3
        inv3 = bg_ref[6] / jnp.sqrt(var3 + _EPS)
        pooled = mx_ref[...]
        out_ref[...] = jnp.maximum(
            (pooled - mean3[None, :]) * inv3[None, :] + bg_ref[7][None, :], 0.0)


def _run_layer3(z2, st2, w2_pad, bg2, tiles=32):
    rows = _B * _S * _K // tiles
    return pl.pallas_call(
        _layer3_body,
        grid=(tiles,),
        in_specs=[
            pl.BlockSpec((rows, 128), lambda t: (t, 0)),
            pl.BlockSpec((2, 128), lambda t: (0, 0)),
            pl.BlockSpec((128, 128), lambda t: (0, 0)),
            pl.BlockSpec((8, 128), lambda t: (0, 0)),
        ],
        out_specs=pl.BlockSpec((_B * _S, 128), lambda t: (0, 0)),
        out_shape=jax.ShapeDtypeStruct((_B * _S, 128), jnp.float32),
        scratch_shapes=[
            pltpu.VMEM((2, 128), jnp.float32),
            pltpu.VMEM((_B * _S, 128), jnp.float32),
        ],
        interpret=_INTERPRET,
    )(z2, st2, w2_pad, bg2)


_NW = 32
_CPW = (_B * _S) // _NW


_NBUF = 4


def _sc_body(spos_hbm, aux_hbm, a_hbm, g_hbm, *refs):
    drows = refs[0:4]
    gbufs = refs[4:8]
    rowss = refs[8:12]
    aux_v = refs[12]
    selbuf = refs[13]
    dsems = refs[14:18]
    gsems = refs[18:22]
    osems = refs[22:26]
    cid = lax.axis_index("c")
    sid = lax.axis_index("s")
    wid = sid * 2 + cid
    base = wid * _CPW
    iota16 = lax.iota(jnp.int32, 16)

    pltpu.sync_copy(aux_hbm.at[pl.ds(base * 128, _CPW * 128)], aux_v)

    def select(drow, gbuf, i, s):
        for r in range(16):
            for l in range(8):
                sp16 = drow[r, pl.ds(l * 16, 16)]
                vals = iota16 + (r * 128 + l * 16)
                plsc.store_scatter(selbuf, [sp16], vals)
        v16 = aux_v[pl.ds(i * 128, 16)]
        cnt = lax.squeeze(lax.slice(v16, (0,), (1,)), (0,))
        first = lax.squeeze(lax.slice(v16, (1,), (2,)), (0,))
        off = lax.shift_right_logical(s, 9) * _N
        for q in range(4):
            cur = selbuf[pl.ds(q * 16, 16)]
            sel = jnp.where((iota16 + q * 16) < cnt, cur, first)
            gbuf[pl.ds(q * 16, 16)] = sel + off

    for p in range(_NBUF - 1):
        pltpu.make_async_copy(spos_hbm.at[base + p], drows[p],
                              dsems[p]).start()

    def outer(gi, carry):
        for par in range(_NBUF):
            i = gi * _NBUF + par
            s = base + i
            pltpu.make_async_copy(spos_hbm.at[s], drows[par],
                                  dsems[par]).wait()

            @pl.when(i + _NBUF - 1 < _CPW)
            def _start_next_dist():
                nx = (par + _NBUF - 1) % _NBUF
                pltpu.make_async_copy(spos_hbm.at[s + _NBUF - 1],
                                      drows[nx], dsems[nx]).start()

            select(drows[par], gbufs[par], i, s)

            @pl.when(i >= _NBUF)
            def _wait_out_copy():
                pltpu.make_async_copy(rowss[par], g_hbm.at[s - _NBUF],
                                      osems[par]).wait()

            pltpu.make_async_copy(a_hbm.at[gbufs[par]], rowss[par],
                                  gsems[par]).start()

            @pl.when(i >= _NBUF - 1)
            def _drain_old_gather():
                pv = (par + 1) % _NBUF
                pltpu.make_async_copy(a_hbm.at[gbufs[pv]],
                                      rowss[pv], gsems[pv]).wait()
                pltpu.make_async_copy(rowss[pv], g_hbm.at[s - (_NBUF - 1)],
                                      osems[pv]).start()
        return carry

    lax.fori_loop(0, _CPW // _NBUF, outer, 0)

    last = base + _CPW - 1
    for d in range(_NBUF - 2, -1, -1):
        p = (_CPW - 1 - d) % _NBUF
        pltpu.make_async_copy(a_hbm.at[gbufs[p]], rowss[p], gsems[p]).wait()
        pltpu.make_async_copy(rowss[p], g_hbm.at[last - d], osems[p]).start()
    for d in range(_NBUF - 1, -1, -1):
        p = (_CPW - 1 - d) % _NBUF
        pltpu.make_async_copy(rowss[p], g_hbm.at[last - d], osems[p]).wait()


def _select_gather(spos, aux_flat, a_pad):
    mesh = plsc.VectorSubcoreMesh(core_axis_name="c", subcore_axis_name="s")
    fn = functools.partial(
        pl.kernel,
        out_type=jax.ShapeDtypeStruct((_B * _S, _K, 128), jnp.float32),
        mesh=mesh,
        compiler_params=pltpu.CompilerParams(needs_layout_passes=False),
        scratch_types=(
            [pltpu.VMEM((16, 128), jnp.int32)] * _NBUF
            + [pltpu.VMEM((_K,), jnp.int32)] * _NBUF
            + [pltpu.VMEM((_K, 128), jnp.float32)] * _NBUF
            + [pltpu.VMEM((_CPW * 128,), jnp.int32),
               pltpu.VMEM((80,), jnp.int32)]
            + [pltpu.SemaphoreType.DMA] * (3 * _NBUF)
        ),
    )(_sc_body)
    return fn(spos, aux_flat, a_pad)


def _pad_w(w, out_lanes=128):
    o, c = w.shape
    wt = jnp.zeros((128, out_lanes), jnp.float32)
    return wt.at[:c, :o].set(w.T)


def kernel(xyz, points, W0, b0, g0, be0, W1, b1, g1, be1, W2, b2, g2, be2):
    xs = xyz[..., 0]
    ys = xyz[..., 1]
    zs = xyz[..., 2]
    cx, cy, cz = _run_fps(xs, ys, zs)
    new_xyz = jnp.stack([cx, cy, cz], axis=-1)

    w0p_pad = _pad_w(W0[:, :64])
    w0x_pad = jnp.zeros((8, 128), jnp.float32).at[:3, :64].set(W0[:, 64:].T)
    b0_pad = jnp.zeros((1, 128), jnp.float32).at[0, :64].set(b0)
    pts_flat = points.reshape(_B * _N, 64)
    pts_pad = jnp.concatenate(
        [pts_flat, jnp.zeros((_B * _N, 64), jnp.float32)], axis=1)
    xyz_flat8 = jnp.concatenate(
        [xyz.reshape(_B * _N, 3), jnp.zeros((_B * _N, 5), jnp.float32)], axis=1)
    nxyz_flat8 = jnp.concatenate(
        [new_xyz.reshape(_B * _S, 3), jnp.zeros((_B * _S, 5), jnp.float32)],
        axis=1)
    a_pad, c_pad = _run_dense(pts_pad, xyz_flat8, nxyz_flat8,
                              w0p_pad, w0x_pad, b0_pad)

    tri = jnp.asarray(np.triu(np.ones((128, 128), np.float32), 1))
    spos, aux, st1 = _run_select(new_xyz, xyz, a_pad, c_pad, tri)
    g = _select_gather(spos, aux.reshape(-1), a_pad)

    bg1 = jnp.zeros((4, 128), jnp.float32)
    bg1 = bg1.at[0, :64].set(b1).at[2, :64].set(g0).at[3, :64].set(be0)
    bg2 = jnp.zeros((8, 128), jnp.float32)
    bg2 = bg2.at[0, :].set(b2).at[2, :64].set(g1).at[3, :64].set(be1)
    bg2 = bg2.at[6, :].set(g2).at[7, :].set(be2)
    z2, st2 = _run_layer2(g, c_pad, st1, _pad_w(W1), bg1)
    new_points = _run_layer3(z2, st2, _pad_w(W2), bg2)

    return new_xyz, new_points.reshape(_B, _S, 128)
